# baseline (speedup 1.0000x reference)
import jax
import jax.numpy as jnp
from jax import lax
from jax.experimental import pallas as pl
from jax.experimental.pallas import tpu as pltpu

_Q = 32
_D = 512
_M = 100000
_BM = 10240
_K = 5
_SB = 1024
_NSETS = 2


def _knn_kernel(obs_ref, W_ref, b_ref, mem_ref, out_ref, a_s, emb_s, run_s):
    i = pl.program_id(0)
    nb = pl.num_programs(0)

    @pl.when(i == 0)
    def _init():
        emb = lax.dot_general(
            obs_ref[...], W_ref[...], (((1,), (0,)), ((), ())),
            preferred_element_type=jnp.float32) + b_ref[...]
        emb_s[...] = emb
        a_s[:, :_D] = (-2.0 * emb).astype(jnp.bfloat16)
        a_s[:, _D:] = jnp.ones((_Q, _D), jnp.bfloat16)
        run_s[...] = jnp.full((_Q, _NSETS * _K * 128), jnp.inf, jnp.float32)

    r = [run_s[:, k * 128:(k + 1) * 128] for k in range(_NSETS * _K)]
    valid = _M - i * _BM
    a = a_s[...]
    iota = lax.broadcasted_iota(jnp.int32, (_Q, _SB), 1)
    for g in range(_BM // _SB):
        mem_g = mem_ref[pl.ds(g * _SB, _SB), :]
        memb = mem_g.astype(jnp.bfloat16)
        msq = memb * memb
        bmat = jnp.concatenate([memb, msq], axis=1)
        t = lax.dot_general(a, bmat, (((1,), (1,)), ((), ())),
                            preferred_element_type=jnp.float32)
        t = jnp.where(iota < valid - g * _SB, t, jnp.inf)
        for c in range(_SB // 128):
            x = t[:, c * 128:(c + 1) * 128]
            o = (c % 2) * _K
            for k in range(_K):
                lo = jnp.minimum(r[o + k], x)
                x = jnp.maximum(r[o + k], x)
                r[o + k] = lo
    for k in range(_NSETS * _K):
        run_s[:, k * 128:(k + 1) * 128] = r[k]

    @pl.when(i == nb - 1)
    def _fin():
        e = emb_s[...]
        q2 = jnp.sum(e * e, axis=1, keepdims=True)
        cand = run_s[...]
        acc = jnp.zeros((_Q, 1), jnp.float32)
        for _ in range(_K):
            m = jnp.min(cand, axis=1, keepdims=True)
            cand = jnp.where(cand == m, jnp.inf, cand)
            acc = acc + jnp.sqrt(jnp.maximum(m + q2, 0.0) + 1e-12)
        out_ref[0, 0] = jnp.sum(acc) / (_Q * _K)


def kernel(obs, memory, W, b):
    nb = pl.cdiv(_M, _BM)
    b2 = b.reshape(1, _D)
    out = pl.pallas_call(
        _knn_kernel,
        grid=(nb,),
        in_specs=[
            pl.BlockSpec(obs.shape, lambda i: (0, 0)),
            pl.BlockSpec(W.shape, lambda i: (0, 0)),
            pl.BlockSpec((1, _D), lambda i: (0, 0)),
            pl.BlockSpec((_BM, _D), lambda i: (i, 0)),
        ],
        out_specs=pl.BlockSpec((1, 1), lambda i: (0, 0),
                               memory_space=pltpu.SMEM),
        out_shape=jax.ShapeDtypeStruct((1, 1), jnp.float32),
        scratch_shapes=[
            pltpu.VMEM((_Q, 2 * _D), jnp.bfloat16),
            pltpu.VMEM((_Q, _D), jnp.float32),
            pltpu.VMEM((_Q, _NSETS * _K * 128), jnp.float32),
        ],
    )(obs, W, b2, memory)
    return out[0, 0]
